# Initial kernel scaffold; baseline (speedup 1.0000x reference)
import jax
import jax.numpy as jnp
from jax.experimental import pallas as pl


def kernel(input, heads, W_dt, U_dt, b_dt, W_td, U_td, b_td):
    raise NotImplementedError("write your pallas kernel here")


# speedup vs baseline: 15.0457x; 15.0457x over previous
import jax
import jax.numpy as jnp
from jax.experimental import pallas as pl
from jax.experimental.pallas import tpu as pltpu

L, B, D, H = 256, 64, 512, 512
H3 = 3 * H
C = 16
NB = L // C


def _gru(gx, gh, bias, hp):
    r = jax.nn.sigmoid(gx[:, :H] + bias[:, :H] + gh[:, :H])
    z = jax.nn.sigmoid(gx[:, H:2 * H] + bias[:, H:2 * H] + gh[:, H:2 * H])
    n = jnp.tanh(gx[:, 2 * H:] + bias[:, 2 * H:] + r * gh[:, 2 * H:])
    return (1.0 - z) * n + z * hp


def _dt_kernel(heads_sref, emb_ref, w_ref, u_ref, b_ref, out_ref, cs_ref, gx_ref):
    i = pl.program_id(0)

    @pl.when(i == 0)
    def _():
        cs_ref[...] = jnp.zeros_like(cs_ref)

    e = emb_ref[...].reshape(C * B, D)
    gx_ref[...] = jnp.dot(e, w_ref[...], preferred_element_type=jnp.float32)

    base = (NB - 1 - i) * C
    bias = b_ref[...]
    u = u_ref[...]

    def step(j, carry):
        l = C - 1 - j
        t = base + l
        hp = cs_ref[t]
        gh = jnp.dot(hp, u, preferred_element_type=jnp.float32)
        gx = gx_ref[pl.ds(l * B, B), :]
        h = _gru(gx, gh, bias, hp)
        out_ref[l] = h
        for b in range(B):
            p = heads_sref[b, t]
            cs_ref[p, b, :] = cs_ref[p, b, :] + h[b, :]
        return carry

    jax.lax.fori_loop(0, C, step, 0)


def _td_kernel(heads_sref, emb_ref, w_ref, u_ref, b_ref, out_ref, hid_ref, gx_ref, hp_ref):
    i = pl.program_id(0)

    @pl.when(i == 0)
    def _():
        hid_ref[L] = jnp.zeros((B, H), jnp.float32)

    e = emb_ref[...].reshape(C * B, D)
    gx_ref[...] = jnp.dot(e, w_ref[...], preferred_element_type=jnp.float32)

    base = i * C
    bias = b_ref[...]
    u = u_ref[...]

    def step(l, carry):
        t = base + l
        for b in range(B):
            p = heads_sref[b, t]
            hp_ref[b, :] = hid_ref[p, b, :]
        hp = hp_ref[...]
        gh = jnp.dot(hp, u, preferred_element_type=jnp.float32)
        gx = gx_ref[pl.ds(l * B, B), :]
        h = _gru(gx, gh, bias, hp)
        out_ref[l] = h
        hid_ref[t] = h
        return carry

    jax.lax.fori_loop(0, C, step, 0)


def _run_pass(body, heads, emb, W, U, bias2, reverse, extra_scratch):
    if reverse:
        blk = lambda i, hr: (NB - 1 - i, 0, 0)
    else:
        blk = lambda i, hr: (i, 0, 0)
    scratch = [
        pltpu.VMEM((L + 1, B, H), jnp.float32),
        pltpu.VMEM((C * B, H3), jnp.float32),
    ] + extra_scratch
    spec = pltpu.PrefetchScalarGridSpec(
        num_scalar_prefetch=1,
        grid=(NB,),
        in_specs=[
            pl.BlockSpec((C, B, D), blk),
            pl.BlockSpec((D, H3), lambda i, hr: (0, 0)),
            pl.BlockSpec((H, H3), lambda i, hr: (0, 0)),
            pl.BlockSpec((1, H3), lambda i, hr: (0, 0)),
        ],
        out_specs=pl.BlockSpec((C, B, H), blk),
        scratch_shapes=scratch,
    )
    return pl.pallas_call(
        body,
        grid_spec=spec,
        out_shape=jax.ShapeDtypeStruct((L, B, H), jnp.float32),
        compiler_params=pltpu.CompilerParams(
            dimension_semantics=("arbitrary",)),
    )(heads, emb, W, U, bias2)


def kernel(input, heads, W_dt, U_dt, b_dt, W_td, U_td, b_td):
    heads_i = heads.astype(jnp.int32)
    dt_hid = _run_pass(_dt_kernel, heads_i, input, W_dt, U_dt,
                       b_dt.reshape(1, H3), True, [])
    td_hid = _run_pass(_td_kernel, heads_i, input, W_td, U_td,
                       b_td.reshape(1, H3), False,
                       [pltpu.VMEM((B, H), jnp.float32)])
    outputs = jnp.concatenate([dt_hid, td_hid], axis=2).transpose(1, 0, 2)
    output_t = jnp.concatenate([dt_hid[0], td_hid[L - 1]], axis=1)[None]
    return outputs, output_t
